# baseline (speedup 1.0000x reference)
import jax
import jax.numpy as jnp
from jax import lax
from jax.experimental import pallas as pl
from jax.experimental.pallas import tpu as pltpu
from jax.experimental.pallas import tpu_sc as plsc

N = 50000
E = 800000
H = 64
VIN = 4
GFS = 16
A = 8
B = 64

NP = 51200
EP = 819200
EC = EP // 128
R = 6400
NBLK = NP // R
CPT = EC // 16
SLICE = NP // 16
F = 32


def _d1_body(x_ref, we_ref, be_ref, w1_ref, out_ref):
    xb = x_ref[...]
    h = jnp.dot(xb, we_ref[...].T, preferred_element_type=jnp.float32)
    h = jnp.maximum(h + be_ref[...], 0.0)
    t = jnp.dot(h, w1_ref[...].T, preferred_element_type=jnp.float32)
    out_ref[...] = jnp.stack([t[:, :F], t[:, F:]], axis=0)


def _d1(x_pad, W_emb, b_emb, W1):
    return pl.pallas_call(
        _d1_body,
        grid=(NBLK,),
        in_specs=[
            pl.BlockSpec((R, VIN), lambda i: (i, 0)),
            pl.BlockSpec((H, VIN), lambda i: (0, 0)),
            pl.BlockSpec((1, H), lambda i: (0, 0)),
            pl.BlockSpec((H, H), lambda i: (0, 0)),
        ],
        out_specs=pl.BlockSpec((2, R, F), lambda i: (0, i, 0)),
        out_shape=jax.ShapeDtypeStruct((2, NP, F), jnp.float32),
    )(x_pad, W_emb, b_emb, W1)


def _d2_body(acc_ref, t_ref, dsq_ref, b_ref, w2_ref, out_ref):
    a = acc_ref[...] + t_ref[...] * dsq_ref[...][None]
    h = jnp.concatenate([a[0], a[1]], axis=1) + b_ref[...]
    h = jnp.maximum(h, 0.0)
    t = jnp.dot(h, w2_ref[...].T, preferred_element_type=jnp.float32)
    out_ref[...] = jnp.stack([t[:, :F], t[:, F:]], axis=0)


def _d2(acc1, t1, dissq, b1, W2):
    return pl.pallas_call(
        _d2_body,
        grid=(NBLK,),
        in_specs=[
            pl.BlockSpec((2, R, F), lambda i: (0, i, 0)),
            pl.BlockSpec((2, R, F), lambda i: (0, i, 0)),
            pl.BlockSpec((R, F), lambda i: (i, 0)),
            pl.BlockSpec((1, H), lambda i: (0, 0)),
            pl.BlockSpec((H, H), lambda i: (0, 0)),
        ],
        out_specs=pl.BlockSpec((2, R, F), lambda i: (0, i, 0)),
        out_shape=jax.ShapeDtypeStruct((2, NP, F), jnp.float32),
    )(acc1, t1, dissq, b1, W2)


def _d3_body(acc_ref, t_ref, dsq_ref, b2_ref, batch_ref, gf_ref, wg_ref,
             bg_ref, wh1_ref, bh1_ref, wh2_ref, bh2_ref, q_ref, sums_ref,
             cnt_ref):
    i = pl.program_id(0)

    @pl.when(i == 0)
    def _init():
        sums_ref[...] = jnp.zeros_like(sums_ref)
        cnt_ref[...] = jnp.zeros_like(cnt_ref)

    a = acc_ref[...] + t_ref[...] * dsq_ref[...][None]
    h = jnp.concatenate([a[0], a[1]], axis=1) + b2_ref[...]
    h = jnp.maximum(h, 0.0)
    bt = batch_ref[0, 0, :]
    ids = lax.broadcasted_iota(jnp.int32, (B, R), 0)
    onehot = (bt[None, :] == ids).astype(jnp.float32)
    sums_ref[...] += jnp.dot(onehot, h, preferred_element_type=jnp.float32)
    cnt_ref[...] += jnp.sum(onehot, axis=1, keepdims=True)

    @pl.when(i == NBLK - 1)
    def _head():
        ge = sums_ref[...] / jnp.maximum(cnt_ref[...], 1.0)
        glob = jnp.dot(gf_ref[...], wg_ref[...].T, preferred_element_type=jnp.float32)
        glob = jnp.maximum(glob + bg_ref[...], 0.0)
        wh1 = wh1_ref[...]
        hid = (jnp.dot(ge, wh1[:, :H].T, preferred_element_type=jnp.float32)
               + jnp.dot(glob, wh1[:, H:].T, preferred_element_type=jnp.float32)
               + bh1_ref[...])
        hid = jnp.maximum(hid, 0.0)
        q_ref[...] = (jnp.dot(hid, wh2_ref[...].T, preferred_element_type=jnp.float32)
                      + bh2_ref[...])


def _d3(acc2, t2, dissq, b2, batch3, gf, Wg, bg, Wh1, bh1, Wh2, bh2):
    return pl.pallas_call(
        _d3_body,
        grid=(NBLK,),
        in_specs=[
            pl.BlockSpec((2, R, F), lambda i: (0, i, 0)),
            pl.BlockSpec((2, R, F), lambda i: (0, i, 0)),
            pl.BlockSpec((R, F), lambda i: (i, 0)),
            pl.BlockSpec((1, H), lambda i: (0, 0)),
            pl.BlockSpec((1, 1, R), lambda i: (i, 0, 0)),
            pl.BlockSpec((B, GFS), lambda i: (0, 0)),
            pl.BlockSpec((H, GFS), lambda i: (0, 0)),
            pl.BlockSpec((1, H), lambda i: (0, 0)),
            pl.BlockSpec((H, 2 * H), lambda i: (0, 0)),
            pl.BlockSpec((1, H), lambda i: (0, 0)),
            pl.BlockSpec((A, H), lambda i: (0, 0)),
            pl.BlockSpec((1, A), lambda i: (0, 0)),
        ],
        out_specs=pl.BlockSpec((B, A), lambda i: (0, 0)),
        out_shape=jax.ShapeDtypeStruct((B, A), jnp.float32),
        scratch_shapes=[
            pltpu.VMEM((B, H), jnp.float32),
            pltpu.VMEM((B, 1), jnp.float32),
        ],
    )(acc2, t2, dissq, b2, batch3, gf, Wg, bg, Wh1, bh1, Wh2, bh2)


def _fori(n, body):
    lax.fori_loop(0, n, lambda i, c: (body(i), 0)[1], 0)


CPT2 = EC // 32


def _s1a_body(cols_ref, ews_ref, pdeg_ref,
              cbA, cbB, wbA, wbB, cxA, cxB, sxA, sxB, dslice, deg_sh,
              semEA, semEB, semSA, semSB):
    c = lax.axis_index("c")
    w = lax.axis_index("s")
    zero16 = jnp.zeros((16,), jnp.float32)
    base = c * (EC // 2) + w * CPT2
    last = base + CPT2 - 1

    def chunk_of(ref, cid):
        return ref.at[pl.ds(cid * 128, 128)]

    def zrow(k):
        dslice[pl.ds(k * 16, 16)] = zero16
    _fori(SLICE // 16, zrow)
    pltpu.sync_copy(dslice, deg_sh.at[pl.ds(w * SLICE, SLICE)])
    plsc.subcore_barrier()

    pltpu.async_copy(chunk_of(cols_ref, base), cbA, semEA)
    pltpu.async_copy(chunk_of(ews_ref, base), wbA, semEA)
    pltpu.async_copy(chunk_of(cols_ref, base + 1), cbB, semEB)
    pltpu.async_copy(chunk_of(ews_ref, base + 1), wbB, semEB)

    def p1half(j, cid, cb, wb, cx, sx, semE, semS):
        pltpu.make_async_copy(chunk_of(cols_ref, cid), cb, semE).wait()
        pltpu.make_async_copy(chunk_of(ews_ref, cid), wb, semE).wait()

        @pl.when(j > 0)
        def _():
            pltpu.make_async_copy(sx, deg_sh.at[cx], semS).wait()
        for k in range(8):
            cx[pl.ds(k * 16, 16)] = cb[pl.ds(k * 16, 16)]
            sx[pl.ds(k * 16, 16)] = wb[pl.ds(k * 16, 16)]
        pltpu.async_copy(sx, deg_sh.at[cx], semS, add=True)
        nxt = jnp.minimum(cid + 2, last)
        pltpu.async_copy(chunk_of(cols_ref, nxt), cb, semE)
        pltpu.async_copy(chunk_of(ews_ref, nxt), wb, semE)

    def p1(j):
        a = base + 2 * j
        p1half(j, a, cbA, wbA, cxA, sxA, semEA, semSA)
        p1half(j, a + 1, cbB, wbB, cxB, sxB, semEB, semSB)
    _fori(CPT2 // 2, p1)
    pltpu.make_async_copy(chunk_of(cols_ref, 0), cbA, semEA).wait()
    pltpu.make_async_copy(chunk_of(ews_ref, 0), wbA, semEA).wait()
    pltpu.make_async_copy(chunk_of(cols_ref, 0), cbB, semEB).wait()
    pltpu.make_async_copy(chunk_of(ews_ref, 0), wbB, semEB).wait()
    pltpu.make_async_copy(sxA, deg_sh.at[cxA], semSA).wait()
    pltpu.make_async_copy(sxB, deg_sh.at[cxB], semSB).wait()
    plsc.subcore_barrier()
    pltpu.sync_copy(deg_sh.at[pl.ds(w * SLICE, SLICE)],
                    pdeg_ref.at[c, pl.ds(w * SLICE, SLICE)])


def _s1a(cols, ews):
    return pl.kernel(
        _s1a_body,
        out_type=jax.ShapeDtypeStruct((2, NP), jnp.float32),
        mesh=plsc.VectorSubcoreMesh(core_axis_name="c", subcore_axis_name="s"),
        compiler_params=pltpu.CompilerParams(needs_layout_passes=False, use_tc_tiling_on_sc=False),
        scratch_types=[
            pltpu.VMEM((128,), jnp.int32),
            pltpu.VMEM((128,), jnp.int32),
            pltpu.VMEM((128,), jnp.float32),
            pltpu.VMEM((128,), jnp.float32),
            pltpu.VMEM((128,), jnp.int32),
            pltpu.VMEM((128,), jnp.int32),
            pltpu.VMEM((128,), jnp.float32),
            pltpu.VMEM((128,), jnp.float32),
            pltpu.VMEM((SLICE,), jnp.float32),
            pltpu.VMEM_SHARED((NP,), jnp.float32),
            pltpu.SemaphoreType.DMA,
            pltpu.SemaphoreType.DMA,
            pltpu.SemaphoreType.DMA,
            pltpu.SemaphoreType.DMA,
        ],
    )(cols, ews)


def _s1b_body(rows_ref, cols_ref, ews_ref, pdeg_ref, norm_ref, dsq_ref,
              rbA, rbB, cbA, cbB, wbA, wbB, nbA, nbB, d2b, dslice, pdg2,
              disv, deg_sh, semEA, semEB, semN):
    c = lax.axis_index("c")
    w = lax.axis_index("s")
    zero16 = jnp.zeros((16,), jnp.float32)
    base = c * (EC // 2) + w * CPT2
    last = base + CPT2 - 1

    def chunk_of(ref, cid):
        return ref.at[pl.ds(cid * 128, 128)]

    pltpu.sync_copy(pdeg_ref.at[0, pl.ds(w * SLICE, SLICE)], dslice)
    pltpu.sync_copy(pdeg_ref.at[1, pl.ds(w * SLICE, SLICE)], pdg2)
    magic = jnp.int32(0x5F3759DF)

    def rsq(k):
        d = dslice[pl.ds(k * 16, 16)] + pdg2[pl.ds(k * 16, 16)] + 1.0
        bits = lax.bitcast_convert_type(d, jnp.int32)
        y = lax.bitcast_convert_type(magic - lax.shift_right_logical(bits, 1), jnp.float32)
        hd = 0.5 * d
        for _ in range(3):
            y = y * (1.5 - hd * y * y)
        dslice[pl.ds(k * 16, 16)] = y
    _fori(SLICE // 16, rsq)
    pltpu.sync_copy(dslice, deg_sh.at[pl.ds(w * SLICE, SLICE)])

    @pl.when(c == 0)
    def _d2emit():
        def d2chunk(j):
            for k in range(8):
                d16 = dslice[pl.ds(j * 128 + k * 16, 16)]
                s16 = d16 * d16
                for m in range(16):
                    e = k * 16 + m
                    sv = s16[m]
                    d2b[e, pl.ds(0, 16)] = zero16 + sv
                    d2b[e, pl.ds(16, 16)] = zero16 + sv
            pltpu.sync_copy(d2b, dsq_ref.at[pl.ds(w * SLICE + j * 128, 128)])
        _fori(SLICE // 128, d2chunk)
    plsc.subcore_barrier()

    pltpu.async_copy(chunk_of(rows_ref, base), rbA, semEA)
    pltpu.async_copy(chunk_of(cols_ref, base), cbA, semEA)
    pltpu.async_copy(chunk_of(ews_ref, base), wbA, semEA)
    pltpu.async_copy(chunk_of(rows_ref, base + 1), rbB, semEB)
    pltpu.async_copy(chunk_of(cols_ref, base + 1), cbB, semEB)
    pltpu.async_copy(chunk_of(ews_ref, base + 1), wbB, semEB)
    pltpu.sync_copy(deg_sh, disv)

    def p3half(j, cid, rb, cb, wb, nb, semE):
        pltpu.make_async_copy(chunk_of(rows_ref, cid), rb, semE).wait()
        pltpu.make_async_copy(chunk_of(cols_ref, cid), cb, semE).wait()
        pltpu.make_async_copy(chunk_of(ews_ref, cid), wb, semE).wait()

        @pl.when(j > 0)
        def _():
            pltpu.make_async_copy(nb, norm_ref.at[pl.ds(0, 128)], semN).wait()
        for k in range(8):
            r16 = rb[pl.ds(k * 16, 16)]
            c16 = cb[pl.ds(k * 16, 16)]
            w16 = wb[pl.ds(k * 16, 16)]
            dr = plsc.load_gather(disv, [r16])
            dc = plsc.load_gather(disv, [c16])
            nb[pl.ds(k * 16, 16)] = dr * w16 * dc
        pltpu.async_copy(nb, norm_ref.at[pl.ds(cid * 128, 128)], semN)
        nxt = jnp.minimum(cid + 2, last)
        pltpu.async_copy(chunk_of(rows_ref, nxt), rb, semE)
        pltpu.async_copy(chunk_of(cols_ref, nxt), cb, semE)
        pltpu.async_copy(chunk_of(ews_ref, nxt), wb, semE)

    def p3(j):
        a = base + 2 * j
        p3half(j, a, rbA, cbA, wbA, nbA, semEA)
        p3half(j, a + 1, rbB, cbB, wbB, nbB, semEB)
    _fori(CPT2 // 2, p3)
    pltpu.make_async_copy(chunk_of(rows_ref, 0), rbA, semEA).wait()
    pltpu.make_async_copy(chunk_of(cols_ref, 0), cbA, semEA).wait()
    pltpu.make_async_copy(chunk_of(ews_ref, 0), wbA, semEA).wait()
    pltpu.make_async_copy(chunk_of(rows_ref, 0), rbB, semEB).wait()
    pltpu.make_async_copy(chunk_of(cols_ref, 0), cbB, semEB).wait()
    pltpu.make_async_copy(chunk_of(ews_ref, 0), wbB, semEB).wait()
    pltpu.make_async_copy(nbA, norm_ref.at[pl.ds(0, 128)], semN).wait()
    pltpu.make_async_copy(nbB, norm_ref.at[pl.ds(0, 128)], semN).wait()


def _s1b(rows, cols, ews, pdeg):
    return pl.kernel(
        _s1b_body,
        out_type=(jax.ShapeDtypeStruct((EP,), jnp.float32),
                  jax.ShapeDtypeStruct((NP, F), jnp.float32)),
        mesh=plsc.VectorSubcoreMesh(core_axis_name="c", subcore_axis_name="s"),
        compiler_params=pltpu.CompilerParams(needs_layout_passes=False, use_tc_tiling_on_sc=False),
        scratch_types=[
            pltpu.VMEM((128,), jnp.int32),
            pltpu.VMEM((128,), jnp.int32),
            pltpu.VMEM((128,), jnp.int32),
            pltpu.VMEM((128,), jnp.int32),
            pltpu.VMEM((128,), jnp.float32),
            pltpu.VMEM((128,), jnp.float32),
            pltpu.VMEM((128,), jnp.float32),
            pltpu.VMEM((128,), jnp.float32),
            pltpu.VMEM((128, F), jnp.float32),
            pltpu.VMEM((SLICE,), jnp.float32),
            pltpu.VMEM((SLICE,), jnp.float32),
            pltpu.VMEM((NP,), jnp.float32),
            pltpu.VMEM_SHARED((NP,), jnp.float32),
            pltpu.SemaphoreType.DMA,
            pltpu.SemaphoreType.DMA,
            pltpu.SemaphoreType.DMA,
        ],
    )(rows, cols, ews, pdeg)


def _mp_body(t_ref, rows_ref, cols_ref, norm_ref, out_ref,
             rbA, rbB, cbA, cbB, nbA, nbB, gxA, gxB, cxA, cxB, gbA, gbB,
             zbuf, acc_sh,
             semEA, semEB, semGA, semGB, semSA, semSB, semZ):
    c = lax.axis_index("c")
    w = lax.axis_index("s")
    coff = c * NP
    zero16 = jnp.zeros((16,), jnp.float32)
    base = w * CPT
    last = base + CPT - 1

    def chunk_of(ref, cid):
        return ref.at[pl.ds(cid * 128, 128)]

    def n_slice(cid):
        return norm_ref.at[pl.ds(cid * 128, 128)]

    for e in range(128):
        zbuf[e, pl.ds(0, 16)] = zero16
        zbuf[e, pl.ds(16, 16)] = zero16
    zd = [pltpu.async_copy(zbuf, acc_sh.at[pl.ds(w * SLICE + j * 128, 128)], semZ)
          for j in range(SLICE // 128)]
    for d in zd:
        d.wait()
    plsc.subcore_barrier()

    pltpu.async_copy(chunk_of(rows_ref, base), rbA, semEA)
    pltpu.async_copy(chunk_of(cols_ref, base), cbA, semEA)
    pltpu.async_copy(n_slice(base), nbA, semEA)
    pltpu.async_copy(chunk_of(rows_ref, base + 1), rbB, semEB)
    pltpu.async_copy(chunk_of(cols_ref, base + 1), cbB, semEB)
    pltpu.async_copy(n_slice(base + 1), nbB, semEB)

    def arrive(j, cid, rb, cb, nb, gx, cx, gb, semE, semG, semS):
        pltpu.make_async_copy(chunk_of(rows_ref, cid), rb, semE).wait()
        pltpu.make_async_copy(chunk_of(cols_ref, cid), cb, semE).wait()
        pltpu.make_async_copy(n_slice(cid), nb, semE).wait()

        @pl.when(j > 0)
        def _():
            pltpu.make_async_copy(gb, acc_sh.at[cx], semS).wait()
        for k in range(8):
            gx[pl.ds(k * 16, 16)] = rb[pl.ds(k * 16, 16)] + coff
            cx[pl.ds(k * 16, 16)] = cb[pl.ds(k * 16, 16)]
        pltpu.async_copy(t_ref.at[gx], gb, semG)
        nxt = jnp.minimum(cid + 2, last)
        pltpu.async_copy(chunk_of(rows_ref, nxt), rb, semE)
        pltpu.async_copy(chunk_of(cols_ref, nxt), cb, semE)

    def scale_scatter(cid, nb, gx, cx, gb, semE, semG, semS):
        pltpu.make_async_copy(t_ref.at[gx], gb, semG).wait()
        for k in range(8):
            n16 = nb[pl.ds(k * 16, 16)]
            for m in range(16):
                e = k * 16 + m
                sv = n16[m]
                gb[e, pl.ds(0, 16)] = gb[e, pl.ds(0, 16)] * sv
                gb[e, pl.ds(16, 16)] = gb[e, pl.ds(16, 16)] * sv
        pltpu.async_copy(gb, acc_sh.at[cx], semS, add=True)
        pltpu.async_copy(n_slice(jnp.minimum(cid + 2, last)), nb, semE)

    def body(j):
        a = base + 2 * j
        b = a + 1
        arrive(j, a, rbA, cbA, nbA, gxA, cxA, gbA, semEA, semGA, semSA)
        arrive(j, b, rbB, cbB, nbB, gxB, cxB, gbB, semEB, semGB, semSB)
        scale_scatter(a, nbA, gxA, cxA, gbA, semEA, semGA, semSA)
        scale_scatter(b, nbB, gxB, cxB, gbB, semEB, semGB, semSB)
    _fori(CPT // 2, body)

    pltpu.make_async_copy(chunk_of(rows_ref, 0), rbA, semEA).wait()
    pltpu.make_async_copy(chunk_of(cols_ref, 0), cbA, semEA).wait()
    pltpu.make_async_copy(n_slice(0), nbA, semEA).wait()
    pltpu.make_async_copy(chunk_of(rows_ref, 0), rbB, semEB).wait()
    pltpu.make_async_copy(chunk_of(cols_ref, 0), cbB, semEB).wait()
    pltpu.make_async_copy(n_slice(0), nbB, semEB).wait()
    pltpu.make_async_copy(gbA, acc_sh.at[cxA], semSA).wait()
    pltpu.make_async_copy(gbB, acc_sh.at[cxB], semSB).wait()
    plsc.subcore_barrier()

    pltpu.sync_copy(acc_sh.at[pl.ds(w * SLICE, SLICE)],
                    out_ref.at[c, pl.ds(w * SLICE, SLICE)])


def _mp(t_flat, rows, cols, norm):
    return pl.kernel(
        _mp_body,
        out_type=jax.ShapeDtypeStruct((2, NP, F), jnp.float32),
        mesh=plsc.VectorSubcoreMesh(core_axis_name="c", subcore_axis_name="s"),
        compiler_params=pltpu.CompilerParams(needs_layout_passes=False, use_tc_tiling_on_sc=False),
        scratch_types=[
            pltpu.VMEM((128,), jnp.int32),
            pltpu.VMEM((128,), jnp.int32),
            pltpu.VMEM((128,), jnp.int32),
            pltpu.VMEM((128,), jnp.int32),
            pltpu.VMEM((128,), jnp.float32),
            pltpu.VMEM((128,), jnp.float32),
            pltpu.VMEM((128,), jnp.int32),
            pltpu.VMEM((128,), jnp.int32),
            pltpu.VMEM((128,), jnp.int32),
            pltpu.VMEM((128,), jnp.int32),
            pltpu.VMEM((128, F), jnp.float32),
            pltpu.VMEM((128, F), jnp.float32),
            pltpu.VMEM((128, F), jnp.float32),
            pltpu.VMEM_SHARED((NP, F), jnp.float32),
            pltpu.SemaphoreType.DMA,
            pltpu.SemaphoreType.DMA,
            pltpu.SemaphoreType.DMA,
            pltpu.SemaphoreType.DMA,
            pltpu.SemaphoreType.DMA,
            pltpu.SemaphoreType.DMA,
            pltpu.SemaphoreType.DMA,
        ],
    )(t_flat, rows, cols, norm)


def kernel(x, edge_index, batch, global_features, edge_weight,
           W_emb, b_emb, W1, b1, W2, b2, Wg, bg, Wh1, bh1, Wh2, bh2):
    x_pad = jnp.pad(x, ((0, NP - N), (0, 0)))
    batch_pad = jnp.pad(batch, (0, NP - N), constant_values=B)
    batch3 = batch_pad.reshape(NBLK, 1, R)
    pad_e = EP - E
    pad_idx = (jnp.arange(pad_e, dtype=jnp.int32) * 997) % N
    rows = jnp.concatenate([edge_index[0], pad_idx])
    cols = jnp.concatenate([edge_index[1], pad_idx])
    ews = jnp.concatenate([edge_weight, jnp.zeros((pad_e,), jnp.float32)])

    b_emb_r = b_emb.reshape(1, H)
    b1_r = b1.reshape(1, H)
    b2_r = b2.reshape(1, H)
    bg_r = bg.reshape(1, H)
    bh1_r = bh1.reshape(1, H)
    bh2_r = bh2.reshape(1, A)

    t1 = _d1(x_pad, W_emb, b_emb_r, W1)
    pdeg = _s1a(cols, ews)
    norm, dissq = _s1b(rows, cols, ews, pdeg)
    acc1 = _mp(t1.reshape(2 * NP, F), rows, cols, norm)
    t2 = _d2(acc1, t1, dissq, b1_r, W2)
    acc2 = _mp(t2.reshape(2 * NP, F), rows, cols, norm)
    q = _d3(acc2, t2, dissq, b2_r, batch3, global_features, Wg, bg_r,
            Wh1, bh1_r, Wh2, bh2_r)
    return q


# speedup vs baseline: 1.2099x; 1.2099x over previous
import jax
import jax.numpy as jnp
from jax import lax
from jax.experimental import pallas as pl
from jax.experimental.pallas import tpu as pltpu
from jax.experimental.pallas import tpu_sc as plsc

N = 50000
E = 800000
H = 64
VIN = 4
GFS = 16
A = 8
B = 64

NP = 51200
EP = 819200
EC = EP // 128
R = 6400
NBLK = NP // R
CPT = EC // 16
SLICE = NP // 16
F = 32


R4 = R // 4
NP4 = NP // 4


def _interleave(a4):
    parts = []
    for j in range(4):
        parts.append(a4[0][:, 32 * j:32 * j + 32])
        parts.append(a4[1][:, 32 * j:32 * j + 32])
    return jnp.concatenate(parts, axis=1)


def _d1_body(x_ref, we_ref, be_ref, w1a_ref, w1b_ref, out_ref):
    xb = x_ref[...]
    h = jnp.dot(xb, we_ref[...], preferred_element_type=jnp.float32)
    h = jnp.maximum(h + be_ref[...], 0.0)
    t0 = jnp.dot(h, w1a_ref[...], preferred_element_type=jnp.float32)
    t1 = jnp.dot(h, w1b_ref[...], preferred_element_type=jnp.float32)
    out_ref[...] = jnp.stack([t0, t1], axis=0)


def _d1(x4, BDWe, be4, BDW1a, BDW1b):
    return pl.pallas_call(
        _d1_body,
        grid=(NBLK,),
        in_specs=[
            pl.BlockSpec((R4, 4 * VIN), lambda i: (i, 0)),
            pl.BlockSpec((4 * VIN, 4 * H), lambda i: (0, 0)),
            pl.BlockSpec((1, 4 * H), lambda i: (0, 0)),
            pl.BlockSpec((4 * H, 128), lambda i: (0, 0)),
            pl.BlockSpec((4 * H, 128), lambda i: (0, 0)),
        ],
        out_specs=pl.BlockSpec((2, R4, 128), lambda i: (0, i, 0)),
        out_shape=jax.ShapeDtypeStruct((2, NP4, 128), jnp.float32),
    )(x4, BDWe, be4, BDW1a, BDW1b)


def _d2_body(acc_ref, t_ref, dsq_ref, b_ref, w2a_ref, w2b_ref, out_ref):
    a4 = acc_ref[...] + t_ref[...] * dsq_ref[...][None]
    h = _interleave(a4) + b_ref[...]
    h = jnp.maximum(h, 0.0)
    t0 = jnp.dot(h, w2a_ref[...], preferred_element_type=jnp.float32)
    t1 = jnp.dot(h, w2b_ref[...], preferred_element_type=jnp.float32)
    out_ref[...] = jnp.stack([t0, t1], axis=0)


def _d2(acc1, t1, dissq, b4, BDW2a, BDW2b):
    return pl.pallas_call(
        _d2_body,
        grid=(NBLK,),
        in_specs=[
            pl.BlockSpec((2, R4, 128), lambda i: (0, i, 0)),
            pl.BlockSpec((2, R4, 128), lambda i: (0, i, 0)),
            pl.BlockSpec((R4, 128), lambda i: (i, 0)),
            pl.BlockSpec((1, 4 * H), lambda i: (0, 0)),
            pl.BlockSpec((4 * H, 128), lambda i: (0, 0)),
            pl.BlockSpec((4 * H, 128), lambda i: (0, 0)),
        ],
        out_specs=pl.BlockSpec((2, R4, 128), lambda i: (0, i, 0)),
        out_shape=jax.ShapeDtypeStruct((2, NP4, 128), jnp.float32),
    )(acc1, t1, dissq, b4, BDW2a, BDW2b)


def _d3_body(acc_ref, t_ref, dsq_ref, b2_ref, batch_ref, gf_ref, wg_ref,
             bg_ref, wh1_ref, bh1_ref, wh2_ref, bh2_ref, q_ref, sums_ref,
             cnt_ref):
    i = pl.program_id(0)

    @pl.when(i == 0)
    def _init():
        sums_ref[...] = jnp.zeros_like(sums_ref)
        cnt_ref[...] = jnp.zeros_like(cnt_ref)

    a4 = acc_ref[...] + t_ref[...] * dsq_ref[...][None]
    h = _interleave(a4) + b2_ref[...]
    h = jnp.maximum(h, 0.0)
    bt4 = batch_ref[0]
    ids = lax.broadcasted_iota(jnp.int32, (B, R4), 0)
    for j in range(4):
        onehot = (bt4[j][None, :] == ids).astype(jnp.float32)
        sums_ref[...] += jnp.dot(onehot, h[:, H * j:H * j + H],
                                 preferred_element_type=jnp.float32)
        cnt_ref[...] += jnp.sum(onehot, axis=1, keepdims=True)

    @pl.when(i == NBLK - 1)
    def _head():
        ge = sums_ref[...] / jnp.maximum(cnt_ref[...], 1.0)
        glob = jnp.dot(gf_ref[...], wg_ref[...].T, preferred_element_type=jnp.float32)
        glob = jnp.maximum(glob + bg_ref[...], 0.0)
        wh1 = wh1_ref[...]
        hid = (jnp.dot(ge, wh1[:, :H].T, preferred_element_type=jnp.float32)
               + jnp.dot(glob, wh1[:, H:].T, preferred_element_type=jnp.float32)
               + bh1_ref[...])
        hid = jnp.maximum(hid, 0.0)
        q_ref[...] = (jnp.dot(hid, wh2_ref[...].T, preferred_element_type=jnp.float32)
                      + bh2_ref[...])


def _d3(acc2, t2, dissq, b2, batch3, gf, Wg, bg, Wh1, bh1, Wh2, bh2):
    return pl.pallas_call(
        _d3_body,
        grid=(NBLK,),
        in_specs=[
            pl.BlockSpec((2, R4, 128), lambda i: (0, i, 0)),
            pl.BlockSpec((2, R4, 128), lambda i: (0, i, 0)),
            pl.BlockSpec((R4, 128), lambda i: (i, 0)),
            pl.BlockSpec((1, 4 * H), lambda i: (0, 0)),
            pl.BlockSpec((1, 4, R4), lambda i: (i, 0, 0)),
            pl.BlockSpec((B, GFS), lambda i: (0, 0)),
            pl.BlockSpec((H, GFS), lambda i: (0, 0)),
            pl.BlockSpec((1, H), lambda i: (0, 0)),
            pl.BlockSpec((H, 2 * H), lambda i: (0, 0)),
            pl.BlockSpec((1, H), lambda i: (0, 0)),
            pl.BlockSpec((A, H), lambda i: (0, 0)),
            pl.BlockSpec((1, A), lambda i: (0, 0)),
        ],
        out_specs=pl.BlockSpec((B, A), lambda i: (0, 0)),
        out_shape=jax.ShapeDtypeStruct((B, A), jnp.float32),
        scratch_shapes=[
            pltpu.VMEM((B, H), jnp.float32),
            pltpu.VMEM((B, 1), jnp.float32),
        ],
    )(acc2, t2, dissq, b2, batch3, gf, Wg, bg, Wh1, bh1, Wh2, bh2)


def _fori(n, body):
    lax.fori_loop(0, n, lambda i, c: (body(i), 0)[1], 0)


CPT2 = EC // 32


def _s1a_body(cols_ref, ews_ref, pdeg_ref,
              cbA, cbB, wbA, wbB, cxA, cxB, sxA, sxB, dslice, deg_sh,
              semEA, semEB, semSA, semSB):
    c = lax.axis_index("c")
    w = lax.axis_index("s")
    zero16 = jnp.zeros((16,), jnp.float32)
    base = c * (EC // 2) + w * CPT2
    last = base + CPT2 - 1

    def chunk_of(ref, cid):
        return ref.at[pl.ds(cid * 128, 128)]

    def zrow(k):
        dslice[pl.ds(k * 16, 16)] = zero16
    _fori(SLICE // 16, zrow)
    pltpu.sync_copy(dslice, deg_sh.at[pl.ds(w * SLICE, SLICE)])
    plsc.subcore_barrier()

    pltpu.async_copy(chunk_of(cols_ref, base), cbA, semEA)
    pltpu.async_copy(chunk_of(ews_ref, base), wbA, semEA)
    pltpu.async_copy(chunk_of(cols_ref, base + 1), cbB, semEB)
    pltpu.async_copy(chunk_of(ews_ref, base + 1), wbB, semEB)

    def p1half(j, cid, cb, wb, cx, sx, semE, semS):
        pltpu.make_async_copy(chunk_of(cols_ref, cid), cb, semE).wait()
        pltpu.make_async_copy(chunk_of(ews_ref, cid), wb, semE).wait()

        @pl.when(j > 0)
        def _():
            pltpu.make_async_copy(sx, deg_sh.at[cx], semS).wait()
        for k in range(8):
            cx[pl.ds(k * 16, 16)] = cb[pl.ds(k * 16, 16)]
            sx[pl.ds(k * 16, 16)] = wb[pl.ds(k * 16, 16)]
        pltpu.async_copy(sx, deg_sh.at[cx], semS, add=True)
        nxt = jnp.minimum(cid + 2, last)
        pltpu.async_copy(chunk_of(cols_ref, nxt), cb, semE)
        pltpu.async_copy(chunk_of(ews_ref, nxt), wb, semE)

    def p1(j):
        a = base + 2 * j
        p1half(j, a, cbA, wbA, cxA, sxA, semEA, semSA)
        p1half(j, a + 1, cbB, wbB, cxB, sxB, semEB, semSB)
    _fori(CPT2 // 2, p1)
    pltpu.make_async_copy(chunk_of(cols_ref, 0), cbA, semEA).wait()
    pltpu.make_async_copy(chunk_of(ews_ref, 0), wbA, semEA).wait()
    pltpu.make_async_copy(chunk_of(cols_ref, 0), cbB, semEB).wait()
    pltpu.make_async_copy(chunk_of(ews_ref, 0), wbB, semEB).wait()
    pltpu.make_async_copy(sxA, deg_sh.at[cxA], semSA).wait()
    pltpu.make_async_copy(sxB, deg_sh.at[cxB], semSB).wait()
    plsc.subcore_barrier()
    pltpu.sync_copy(deg_sh.at[pl.ds(w * SLICE, SLICE)],
                    pdeg_ref.at[c, pl.ds(w * SLICE, SLICE)])


def _s1a(cols, ews):
    return pl.kernel(
        _s1a_body,
        out_type=jax.ShapeDtypeStruct((2, NP), jnp.float32),
        mesh=plsc.VectorSubcoreMesh(core_axis_name="c", subcore_axis_name="s"),
        compiler_params=pltpu.CompilerParams(needs_layout_passes=False, use_tc_tiling_on_sc=False),
        scratch_types=[
            pltpu.VMEM((128,), jnp.int32),
            pltpu.VMEM((128,), jnp.int32),
            pltpu.VMEM((128,), jnp.float32),
            pltpu.VMEM((128,), jnp.float32),
            pltpu.VMEM((128,), jnp.int32),
            pltpu.VMEM((128,), jnp.int32),
            pltpu.VMEM((128,), jnp.float32),
            pltpu.VMEM((128,), jnp.float32),
            pltpu.VMEM((SLICE,), jnp.float32),
            pltpu.VMEM_SHARED((NP,), jnp.float32),
            pltpu.SemaphoreType.DMA,
            pltpu.SemaphoreType.DMA,
            pltpu.SemaphoreType.DMA,
            pltpu.SemaphoreType.DMA,
        ],
    )(cols, ews)


def _s1b_body(rows_ref, cols_ref, ews_ref, pdeg_ref, norm_ref, dsq_ref,
              rbA, rbB, cbA, cbB, wbA, wbB, nbA, nbB, d2b, dslice, pdg2,
              disv, deg_sh, semEA, semEB, semN):
    c = lax.axis_index("c")
    w = lax.axis_index("s")
    zero16 = jnp.zeros((16,), jnp.float32)
    base = c * (EC // 2) + w * CPT2
    last = base + CPT2 - 1

    def chunk_of(ref, cid):
        return ref.at[pl.ds(cid * 128, 128)]

    pltpu.sync_copy(pdeg_ref.at[0, pl.ds(w * SLICE, SLICE)], dslice)
    pltpu.sync_copy(pdeg_ref.at[1, pl.ds(w * SLICE, SLICE)], pdg2)
    magic = jnp.int32(0x5F3759DF)

    def rsq(k):
        d = dslice[pl.ds(k * 16, 16)] + pdg2[pl.ds(k * 16, 16)] + 1.0
        bits = lax.bitcast_convert_type(d, jnp.int32)
        y = lax.bitcast_convert_type(magic - lax.shift_right_logical(bits, 1), jnp.float32)
        hd = 0.5 * d
        for _ in range(3):
            y = y * (1.5 - hd * y * y)
        dslice[pl.ds(k * 16, 16)] = y
    _fori(SLICE // 16, rsq)
    pltpu.sync_copy(dslice, deg_sh.at[pl.ds(w * SLICE, SLICE)])

    @pl.when(c == 0)
    def _d2emit():
        def d2chunk(j):
            for k in range(8):
                d16 = dslice[pl.ds(j * 128 + k * 16, 16)]
                s16 = d16 * d16
                for m in range(16):
                    e = k * 16 + m
                    sv = s16[m]
                    d2b[e // 4, pl.ds((e % 4) * 32, 16)] = zero16 + sv
                    d2b[e // 4, pl.ds((e % 4) * 32 + 16, 16)] = zero16 + sv
            pltpu.sync_copy(d2b, dsq_ref.at[pl.ds(w * (SLICE // 4) + j * 32, 32)])
        _fori(SLICE // 128, d2chunk)
    plsc.subcore_barrier()

    pltpu.async_copy(chunk_of(rows_ref, base), rbA, semEA)
    pltpu.async_copy(chunk_of(cols_ref, base), cbA, semEA)
    pltpu.async_copy(chunk_of(ews_ref, base), wbA, semEA)
    pltpu.async_copy(chunk_of(rows_ref, base + 1), rbB, semEB)
    pltpu.async_copy(chunk_of(cols_ref, base + 1), cbB, semEB)
    pltpu.async_copy(chunk_of(ews_ref, base + 1), wbB, semEB)
    pltpu.sync_copy(deg_sh, disv)

    def p3half(j, cid, rb, cb, wb, nb, semE):
        pltpu.make_async_copy(chunk_of(rows_ref, cid), rb, semE).wait()
        pltpu.make_async_copy(chunk_of(cols_ref, cid), cb, semE).wait()
        pltpu.make_async_copy(chunk_of(ews_ref, cid), wb, semE).wait()

        @pl.when(j > 0)
        def _():
            pltpu.make_async_copy(nb, norm_ref.at[pl.ds(0, 128)], semN).wait()
        for k in range(8):
            r16 = rb[pl.ds(k * 16, 16)]
            c16 = cb[pl.ds(k * 16, 16)]
            w16 = wb[pl.ds(k * 16, 16)]
            dr = plsc.load_gather(disv, [r16])
            dc = plsc.load_gather(disv, [c16])
            nb[pl.ds(k * 16, 16)] = dr * w16 * dc
        pltpu.async_copy(nb, norm_ref.at[pl.ds(cid * 128, 128)], semN)
        nxt = jnp.minimum(cid + 2, last)
        pltpu.async_copy(chunk_of(rows_ref, nxt), rb, semE)
        pltpu.async_copy(chunk_of(cols_ref, nxt), cb, semE)
        pltpu.async_copy(chunk_of(ews_ref, nxt), wb, semE)

    def p3(j):
        a = base + 2 * j
        p3half(j, a, rbA, cbA, wbA, nbA, semEA)
        p3half(j, a + 1, rbB, cbB, wbB, nbB, semEB)
    _fori(CPT2 // 2, p3)
    pltpu.make_async_copy(chunk_of(rows_ref, 0), rbA, semEA).wait()
    pltpu.make_async_copy(chunk_of(cols_ref, 0), cbA, semEA).wait()
    pltpu.make_async_copy(chunk_of(ews_ref, 0), wbA, semEA).wait()
    pltpu.make_async_copy(chunk_of(rows_ref, 0), rbB, semEB).wait()
    pltpu.make_async_copy(chunk_of(cols_ref, 0), cbB, semEB).wait()
    pltpu.make_async_copy(chunk_of(ews_ref, 0), wbB, semEB).wait()
    pltpu.make_async_copy(nbA, norm_ref.at[pl.ds(0, 128)], semN).wait()
    pltpu.make_async_copy(nbB, norm_ref.at[pl.ds(0, 128)], semN).wait()


def _s1b(rows, cols, ews, pdeg):
    return pl.kernel(
        _s1b_body,
        out_type=(jax.ShapeDtypeStruct((EP,), jnp.float32),
                  jax.ShapeDtypeStruct((NP4, 128), jnp.float32)),
        mesh=plsc.VectorSubcoreMesh(core_axis_name="c", subcore_axis_name="s"),
        compiler_params=pltpu.CompilerParams(needs_layout_passes=False, use_tc_tiling_on_sc=False),
        scratch_types=[
            pltpu.VMEM((128,), jnp.int32),
            pltpu.VMEM((128,), jnp.int32),
            pltpu.VMEM((128,), jnp.int32),
            pltpu.VMEM((128,), jnp.int32),
            pltpu.VMEM((128,), jnp.float32),
            pltpu.VMEM((128,), jnp.float32),
            pltpu.VMEM((128,), jnp.float32),
            pltpu.VMEM((128,), jnp.float32),
            pltpu.VMEM((32, 128), jnp.float32),
            pltpu.VMEM((SLICE,), jnp.float32),
            pltpu.VMEM((SLICE,), jnp.float32),
            pltpu.VMEM((NP,), jnp.float32),
            pltpu.VMEM_SHARED((NP,), jnp.float32),
            pltpu.SemaphoreType.DMA,
            pltpu.SemaphoreType.DMA,
            pltpu.SemaphoreType.DMA,
        ],
    )(rows, cols, ews, pdeg)


def _mp_body(t_ref, rows_ref, cols_ref, norm_ref, out_ref,
             rbA, rbB, cbA, cbB, nbA, nbB, gxA, gxB, cxA, cxB, gbA, gbB,
             zbuf, acc_sh,
             semEA, semEB, semGA, semGB, semSA, semSB, semZ):
    c = lax.axis_index("c")
    w = lax.axis_index("s")
    coff = c * NP
    zero16 = jnp.zeros((16,), jnp.float32)
    base = w * CPT
    last = base + CPT - 1

    def chunk_of(ref, cid):
        return ref.at[pl.ds(cid * 128, 128)]

    def n_slice(cid):
        return norm_ref.at[pl.ds(cid * 128, 128)]

    for e in range(128):
        zbuf[e, pl.ds(0, 16)] = zero16
        zbuf[e, pl.ds(16, 16)] = zero16
    zd = [pltpu.async_copy(zbuf, acc_sh.at[pl.ds(w * SLICE + j * 128, 128)], semZ)
          for j in range(SLICE // 128)]
    for d in zd:
        d.wait()
    plsc.subcore_barrier()

    pltpu.async_copy(chunk_of(rows_ref, base), rbA, semEA)
    pltpu.async_copy(chunk_of(cols_ref, base), cbA, semEA)
    pltpu.async_copy(n_slice(base), nbA, semEA)
    pltpu.async_copy(chunk_of(rows_ref, base + 1), rbB, semEB)
    pltpu.async_copy(chunk_of(cols_ref, base + 1), cbB, semEB)
    pltpu.async_copy(n_slice(base + 1), nbB, semEB)

    def arrive(j, cid, rb, cb, nb, gx, cx, gb, semE, semG, semS):
        pltpu.make_async_copy(chunk_of(rows_ref, cid), rb, semE).wait()
        pltpu.make_async_copy(chunk_of(cols_ref, cid), cb, semE).wait()
        pltpu.make_async_copy(n_slice(cid), nb, semE).wait()

        @pl.when(j > 0)
        def _():
            pltpu.make_async_copy(gb, acc_sh.at[cx], semS).wait()
        for k in range(8):
            gx[pl.ds(k * 16, 16)] = rb[pl.ds(k * 16, 16)] + coff
            cx[pl.ds(k * 16, 16)] = cb[pl.ds(k * 16, 16)]
        pltpu.async_copy(t_ref.at[gx], gb, semG)
        nxt = jnp.minimum(cid + 2, last)
        pltpu.async_copy(chunk_of(rows_ref, nxt), rb, semE)
        pltpu.async_copy(chunk_of(cols_ref, nxt), cb, semE)

    def scale_scatter(cid, nb, gx, cx, gb, semE, semG, semS):
        pltpu.make_async_copy(t_ref.at[gx], gb, semG).wait()
        for k in range(8):
            n16 = nb[pl.ds(k * 16, 16)]
            for m in range(16):
                e = k * 16 + m
                sv = n16[m]
                gb[e, pl.ds(0, 16)] = gb[e, pl.ds(0, 16)] * sv
                gb[e, pl.ds(16, 16)] = gb[e, pl.ds(16, 16)] * sv
        pltpu.async_copy(gb, acc_sh.at[cx], semS, add=True)
        pltpu.async_copy(n_slice(jnp.minimum(cid + 2, last)), nb, semE)

    def body(j):
        a = base + 2 * j
        b = a + 1
        arrive(j, a, rbA, cbA, nbA, gxA, cxA, gbA, semEA, semGA, semSA)
        arrive(j, b, rbB, cbB, nbB, gxB, cxB, gbB, semEB, semGB, semSB)
        scale_scatter(a, nbA, gxA, cxA, gbA, semEA, semGA, semSA)
        scale_scatter(b, nbB, gxB, cxB, gbB, semEB, semGB, semSB)
    _fori(CPT // 2, body)

    pltpu.make_async_copy(chunk_of(rows_ref, 0), rbA, semEA).wait()
    pltpu.make_async_copy(chunk_of(cols_ref, 0), cbA, semEA).wait()
    pltpu.make_async_copy(n_slice(0), nbA, semEA).wait()
    pltpu.make_async_copy(chunk_of(rows_ref, 0), rbB, semEB).wait()
    pltpu.make_async_copy(chunk_of(cols_ref, 0), cbB, semEB).wait()
    pltpu.make_async_copy(n_slice(0), nbB, semEB).wait()
    pltpu.make_async_copy(gbA, acc_sh.at[cxA], semSA).wait()
    pltpu.make_async_copy(gbB, acc_sh.at[cxB], semSB).wait()
    plsc.subcore_barrier()

    pltpu.sync_copy(acc_sh.at[pl.ds(w * SLICE, SLICE)],
                    out_ref.at[c, pl.ds(w * SLICE, SLICE)])


def _mp(t_flat, rows, cols, norm):
    return pl.kernel(
        _mp_body,
        out_type=jax.ShapeDtypeStruct((2, NP, F), jnp.float32),
        mesh=plsc.VectorSubcoreMesh(core_axis_name="c", subcore_axis_name="s"),
        compiler_params=pltpu.CompilerParams(needs_layout_passes=False, use_tc_tiling_on_sc=False),
        scratch_types=[
            pltpu.VMEM((128,), jnp.int32),
            pltpu.VMEM((128,), jnp.int32),
            pltpu.VMEM((128,), jnp.int32),
            pltpu.VMEM((128,), jnp.int32),
            pltpu.VMEM((128,), jnp.float32),
            pltpu.VMEM((128,), jnp.float32),
            pltpu.VMEM((128,), jnp.int32),
            pltpu.VMEM((128,), jnp.int32),
            pltpu.VMEM((128,), jnp.int32),
            pltpu.VMEM((128,), jnp.int32),
            pltpu.VMEM((128, F), jnp.float32),
            pltpu.VMEM((128, F), jnp.float32),
            pltpu.VMEM((128, F), jnp.float32),
            pltpu.VMEM_SHARED((NP, F), jnp.float32),
            pltpu.SemaphoreType.DMA,
            pltpu.SemaphoreType.DMA,
            pltpu.SemaphoreType.DMA,
            pltpu.SemaphoreType.DMA,
            pltpu.SemaphoreType.DMA,
            pltpu.SemaphoreType.DMA,
            pltpu.SemaphoreType.DMA,
        ],
    )(t_flat, rows, cols, norm)


def kernel(x, edge_index, batch, global_features, edge_weight,
           W_emb, b_emb, W1, b1, W2, b2, Wg, bg, Wh1, bh1, Wh2, bh2):
    x_pad = jnp.pad(x, ((0, NP - N), (0, 0)))
    x4 = x_pad.reshape(NP4, 4 * VIN)
    batch_pad = jnp.pad(batch, (0, NP - N), constant_values=B)
    batch3 = batch_pad.reshape(NBLK, R4, 4).transpose(0, 2, 1)

    def bd4(w):
        return jax.scipy.linalg.block_diag(w, w, w, w)

    BDWe = bd4(W_emb.T)
    BDW1a = bd4(W1[:F, :].T)
    BDW1b = bd4(W1[F:, :].T)
    BDW2a = bd4(W2[:F, :].T)
    BDW2b = bd4(W2[F:, :].T)
    be4 = jnp.tile(b_emb, 4).reshape(1, 4 * H)
    b1_4 = jnp.tile(b1, 4).reshape(1, 4 * H)
    b2_4 = jnp.tile(b2, 4).reshape(1, 4 * H)
    pad_e = EP - E
    pad_idx = (jnp.arange(pad_e, dtype=jnp.int32) * 997) % N
    rows = jnp.concatenate([edge_index[0], pad_idx])
    cols = jnp.concatenate([edge_index[1], pad_idx])
    ews = jnp.concatenate([edge_weight, jnp.zeros((pad_e,), jnp.float32)])

    bg_r = bg.reshape(1, H)
    bh1_r = bh1.reshape(1, H)
    bh2_r = bh2.reshape(1, A)

    t1 = _d1(x4, BDWe, be4, BDW1a, BDW1b)
    pdeg = _s1a(cols, ews)
    norm, dissq = _s1b(rows, cols, ews, pdeg)
    acc1 = _mp(t1.reshape(2 * NP, F), rows, cols, norm)
    t2 = _d2(acc1.reshape(2, NP4, 128), t1, dissq, b1_4, BDW2a, BDW2b)
    acc2 = _mp(t2.reshape(2 * NP, F), rows, cols, norm)
    q = _d3(acc2.reshape(2, NP4, 128), t2, dissq, b2_4, batch3,
            global_features, Wg, bg_r, Wh1, bh1_r, Wh2, bh2_r)
    return q


# speedup vs baseline: 1.2883x; 1.0648x over previous
import jax
import jax.numpy as jnp
from jax import lax
from jax.experimental import pallas as pl
from jax.experimental.pallas import tpu as pltpu
from jax.experimental.pallas import tpu_sc as plsc

N = 50000
E = 800000
H = 64
VIN = 4
GFS = 16
A = 8
B = 64

NP = 51200
EP = 819200
EC = EP // 128
R = 6400
NBLK = NP // R
CPT = EC // 16
SLICE = NP // 16
F = 32


R4 = R // 4
NP4 = NP // 4


def _interleave(a4):
    parts = []
    for j in range(4):
        parts.append(a4[0][:, 32 * j:32 * j + 32])
        parts.append(a4[1][:, 32 * j:32 * j + 32])
    return jnp.concatenate(parts, axis=1)


def _d1_body(x_ref, we_ref, be_ref, w1a_ref, w1b_ref, out_ref):
    xb = x_ref[...]
    h = jnp.dot(xb, we_ref[...], preferred_element_type=jnp.float32)
    h = jnp.maximum(h + be_ref[...], 0.0)
    t0 = jnp.dot(h, w1a_ref[...], preferred_element_type=jnp.float32)
    t1 = jnp.dot(h, w1b_ref[...], preferred_element_type=jnp.float32)
    out_ref[...] = jnp.stack([t0, t1], axis=0)


def _d1(x4, BDWe, be4, BDW1a, BDW1b):
    return pl.pallas_call(
        _d1_body,
        grid=(NBLK,),
        in_specs=[
            pl.BlockSpec((R4, 4 * VIN), lambda i: (i, 0)),
            pl.BlockSpec((4 * VIN, 4 * H), lambda i: (0, 0)),
            pl.BlockSpec((1, 4 * H), lambda i: (0, 0)),
            pl.BlockSpec((4 * H, 128), lambda i: (0, 0)),
            pl.BlockSpec((4 * H, 128), lambda i: (0, 0)),
        ],
        out_specs=pl.BlockSpec((2, R4, 128), lambda i: (0, i, 0)),
        out_shape=jax.ShapeDtypeStruct((2, NP4, 128), jnp.float32),
    )(x4, BDWe, be4, BDW1a, BDW1b)


def _d2_body(acc_ref, t_ref, dsq_ref, b_ref, w2a_ref, w2b_ref, out_ref):
    a4 = acc_ref[...] + t_ref[...] * dsq_ref[...][None]
    h = _interleave(a4) + b_ref[...]
    h = jnp.maximum(h, 0.0)
    t0 = jnp.dot(h, w2a_ref[...], preferred_element_type=jnp.float32)
    t1 = jnp.dot(h, w2b_ref[...], preferred_element_type=jnp.float32)
    out_ref[...] = jnp.stack([t0, t1], axis=0)


def _d2(acc1, t1, dissq, b4, BDW2a, BDW2b):
    return pl.pallas_call(
        _d2_body,
        grid=(NBLK,),
        in_specs=[
            pl.BlockSpec((2, R4, 128), lambda i: (0, i, 0)),
            pl.BlockSpec((2, R4, 128), lambda i: (0, i, 0)),
            pl.BlockSpec((R4, 128), lambda i: (i, 0)),
            pl.BlockSpec((1, 4 * H), lambda i: (0, 0)),
            pl.BlockSpec((4 * H, 128), lambda i: (0, 0)),
            pl.BlockSpec((4 * H, 128), lambda i: (0, 0)),
        ],
        out_specs=pl.BlockSpec((2, R4, 128), lambda i: (0, i, 0)),
        out_shape=jax.ShapeDtypeStruct((2, NP4, 128), jnp.float32),
    )(acc1, t1, dissq, b4, BDW2a, BDW2b)


def _d3_body(acc_ref, t_ref, dsq_ref, b2_ref, batch_ref, gf_ref, wg_ref,
             bg_ref, wh1_ref, bh1_ref, wh2_ref, bh2_ref, q_ref, sums_ref,
             cnt_ref):
    i = pl.program_id(0)

    @pl.when(i == 0)
    def _init():
        sums_ref[...] = jnp.zeros_like(sums_ref)
        cnt_ref[...] = jnp.zeros_like(cnt_ref)

    a4 = acc_ref[...] + t_ref[...] * dsq_ref[...][None]
    h = _interleave(a4) + b2_ref[...]
    h = jnp.maximum(h, 0.0)
    bt4 = batch_ref[0]
    ids = lax.broadcasted_iota(jnp.int32, (B, R4), 0)
    for j in range(4):
        onehot = (bt4[j][None, :] == ids).astype(jnp.float32)
        sums_ref[...] += jnp.dot(onehot, h[:, H * j:H * j + H],
                                 preferred_element_type=jnp.float32)
        cnt_ref[...] += jnp.sum(onehot, axis=1, keepdims=True)

    @pl.when(i == NBLK - 1)
    def _head():
        ge = sums_ref[...] / jnp.maximum(cnt_ref[...], 1.0)
        glob = jnp.dot(gf_ref[...], wg_ref[...].T, preferred_element_type=jnp.float32)
        glob = jnp.maximum(glob + bg_ref[...], 0.0)
        wh1 = wh1_ref[...]
        hid = (jnp.dot(ge, wh1[:, :H].T, preferred_element_type=jnp.float32)
               + jnp.dot(glob, wh1[:, H:].T, preferred_element_type=jnp.float32)
               + bh1_ref[...])
        hid = jnp.maximum(hid, 0.0)
        q_ref[...] = (jnp.dot(hid, wh2_ref[...].T, preferred_element_type=jnp.float32)
                      + bh2_ref[...])


def _d3(acc2, t2, dissq, b2, batch3, gf, Wg, bg, Wh1, bh1, Wh2, bh2):
    return pl.pallas_call(
        _d3_body,
        grid=(NBLK,),
        in_specs=[
            pl.BlockSpec((2, R4, 128), lambda i: (0, i, 0)),
            pl.BlockSpec((2, R4, 128), lambda i: (0, i, 0)),
            pl.BlockSpec((R4, 128), lambda i: (i, 0)),
            pl.BlockSpec((1, 4 * H), lambda i: (0, 0)),
            pl.BlockSpec((1, 4, R4), lambda i: (i, 0, 0)),
            pl.BlockSpec((B, GFS), lambda i: (0, 0)),
            pl.BlockSpec((H, GFS), lambda i: (0, 0)),
            pl.BlockSpec((1, H), lambda i: (0, 0)),
            pl.BlockSpec((H, 2 * H), lambda i: (0, 0)),
            pl.BlockSpec((1, H), lambda i: (0, 0)),
            pl.BlockSpec((A, H), lambda i: (0, 0)),
            pl.BlockSpec((1, A), lambda i: (0, 0)),
        ],
        out_specs=pl.BlockSpec((B, A), lambda i: (0, 0)),
        out_shape=jax.ShapeDtypeStruct((B, A), jnp.float32),
        scratch_shapes=[
            pltpu.VMEM((B, H), jnp.float32),
            pltpu.VMEM((B, 1), jnp.float32),
        ],
    )(acc2, t2, dissq, b2, batch3, gf, Wg, bg, Wh1, bh1, Wh2, bh2)


def _fori(n, body):
    lax.fori_loop(0, n, lambda i, c: (body(i), 0)[1], 0)


CPT2 = EC // 32


def _s1a_body(cols_ref, ews_ref, pdeg_ref,
              cbA, cbB, wbA, wbB, cxA, cxB, sxA, sxB, dslice, deg_sh,
              semEA, semEB, semSA, semSB):
    c = lax.axis_index("c")
    w = lax.axis_index("s")
    zero16 = jnp.zeros((16,), jnp.float32)
    base = c * (EC // 2) + w * CPT2
    last = base + CPT2 - 1

    def chunk_of(ref, cid):
        return ref.at[pl.ds(cid * 128, 128)]

    def zrow(k):
        dslice[pl.ds(k * 16, 16)] = zero16
    _fori(SLICE // 16, zrow)
    pltpu.sync_copy(dslice, deg_sh.at[pl.ds(w * SLICE, SLICE)])
    plsc.subcore_barrier()

    pltpu.async_copy(chunk_of(cols_ref, base), cbA, semEA)
    pltpu.async_copy(chunk_of(ews_ref, base), wbA, semEA)
    pltpu.async_copy(chunk_of(cols_ref, base + 1), cbB, semEB)
    pltpu.async_copy(chunk_of(ews_ref, base + 1), wbB, semEB)

    def p1half(j, cid, cb, wb, cx, sx, semE, semS):
        pltpu.make_async_copy(chunk_of(cols_ref, cid), cb, semE).wait()
        pltpu.make_async_copy(chunk_of(ews_ref, cid), wb, semE).wait()

        @pl.when(j > 0)
        def _():
            pltpu.make_async_copy(sx, deg_sh.at[cx], semS).wait()
        for k in range(8):
            cx[pl.ds(k * 16, 16)] = cb[pl.ds(k * 16, 16)]
            sx[pl.ds(k * 16, 16)] = wb[pl.ds(k * 16, 16)]
        pltpu.async_copy(sx, deg_sh.at[cx], semS, add=True)
        nxt = jnp.minimum(cid + 2, last)
        pltpu.async_copy(chunk_of(cols_ref, nxt), cb, semE)
        pltpu.async_copy(chunk_of(ews_ref, nxt), wb, semE)

    def p1(j):
        a = base + 2 * j
        p1half(j, a, cbA, wbA, cxA, sxA, semEA, semSA)
        p1half(j, a + 1, cbB, wbB, cxB, sxB, semEB, semSB)
    _fori(CPT2 // 2, p1)
    pltpu.make_async_copy(chunk_of(cols_ref, 0), cbA, semEA).wait()
    pltpu.make_async_copy(chunk_of(ews_ref, 0), wbA, semEA).wait()
    pltpu.make_async_copy(chunk_of(cols_ref, 0), cbB, semEB).wait()
    pltpu.make_async_copy(chunk_of(ews_ref, 0), wbB, semEB).wait()
    pltpu.make_async_copy(sxA, deg_sh.at[cxA], semSA).wait()
    pltpu.make_async_copy(sxB, deg_sh.at[cxB], semSB).wait()
    plsc.subcore_barrier()
    pltpu.sync_copy(deg_sh.at[pl.ds(w * SLICE, SLICE)],
                    pdeg_ref.at[c, pl.ds(w * SLICE, SLICE)])


def _s1a(cols, ews):
    return pl.kernel(
        _s1a_body,
        out_type=jax.ShapeDtypeStruct((2, NP), jnp.float32),
        mesh=plsc.VectorSubcoreMesh(core_axis_name="c", subcore_axis_name="s"),
        compiler_params=pltpu.CompilerParams(needs_layout_passes=False, use_tc_tiling_on_sc=False),
        scratch_types=[
            pltpu.VMEM((128,), jnp.int32),
            pltpu.VMEM((128,), jnp.int32),
            pltpu.VMEM((128,), jnp.float32),
            pltpu.VMEM((128,), jnp.float32),
            pltpu.VMEM((128,), jnp.int32),
            pltpu.VMEM((128,), jnp.int32),
            pltpu.VMEM((128,), jnp.float32),
            pltpu.VMEM((128,), jnp.float32),
            pltpu.VMEM((SLICE,), jnp.float32),
            pltpu.VMEM_SHARED((NP,), jnp.float32),
            pltpu.SemaphoreType.DMA,
            pltpu.SemaphoreType.DMA,
            pltpu.SemaphoreType.DMA,
            pltpu.SemaphoreType.DMA,
        ],
    )(cols, ews)


def _s1b_body(rows_ref, cols_ref, ews_ref, pdeg_ref, norm_ref, dsq_ref,
              rbA, rbB, cbA, cbB, wbA, wbB, nbA, nbB, d2b, dslice, pdg2,
              disv, deg_sh, semEA, semEB, semN):
    c = lax.axis_index("c")
    w = lax.axis_index("s")
    zero16 = jnp.zeros((16,), jnp.float32)
    base = c * (EC // 2) + w * CPT2
    last = base + CPT2 - 1

    def chunk_of(ref, cid):
        return ref.at[pl.ds(cid * 128, 128)]

    pltpu.sync_copy(pdeg_ref.at[0, pl.ds(w * SLICE, SLICE)], dslice)
    pltpu.sync_copy(pdeg_ref.at[1, pl.ds(w * SLICE, SLICE)], pdg2)
    magic = jnp.int32(0x5F3759DF)

    def rsq(k):
        d = dslice[pl.ds(k * 16, 16)] + pdg2[pl.ds(k * 16, 16)] + 1.0
        bits = lax.bitcast_convert_type(d, jnp.int32)
        y = lax.bitcast_convert_type(magic - lax.shift_right_logical(bits, 1), jnp.float32)
        hd = 0.5 * d
        for _ in range(3):
            y = y * (1.5 - hd * y * y)
        dslice[pl.ds(k * 16, 16)] = y
    _fori(SLICE // 16, rsq)
    pltpu.sync_copy(dslice, deg_sh.at[pl.ds(w * SLICE, SLICE)])

    @pl.when(c == 0)
    def _d2emit():
        def d2chunk(j):
            for k in range(8):
                d16 = dslice[pl.ds(j * 128 + k * 16, 16)]
                s16 = d16 * d16
                for m in range(16):
                    e = k * 16 + m
                    sv = s16[m]
                    d2b[e // 4, pl.ds((e % 4) * 32, 16)] = zero16 + sv
                    d2b[e // 4, pl.ds((e % 4) * 32 + 16, 16)] = zero16 + sv
            pltpu.sync_copy(d2b, dsq_ref.at[pl.ds(w * (SLICE // 4) + j * 32, 32)])
        _fori(SLICE // 128, d2chunk)
    plsc.subcore_barrier()

    pltpu.async_copy(chunk_of(rows_ref, base), rbA, semEA)
    pltpu.async_copy(chunk_of(cols_ref, base), cbA, semEA)
    pltpu.async_copy(chunk_of(ews_ref, base), wbA, semEA)
    pltpu.async_copy(chunk_of(rows_ref, base + 1), rbB, semEB)
    pltpu.async_copy(chunk_of(cols_ref, base + 1), cbB, semEB)
    pltpu.async_copy(chunk_of(ews_ref, base + 1), wbB, semEB)
    pltpu.sync_copy(deg_sh, disv)

    def p3half(j, cid, rb, cb, wb, nb, semE):
        pltpu.make_async_copy(chunk_of(rows_ref, cid), rb, semE).wait()
        pltpu.make_async_copy(chunk_of(cols_ref, cid), cb, semE).wait()
        pltpu.make_async_copy(chunk_of(ews_ref, cid), wb, semE).wait()

        @pl.when(j > 0)
        def _():
            pltpu.make_async_copy(nb, norm_ref.at[pl.ds(0, 128)], semN).wait()
        for k in range(8):
            r16 = rb[pl.ds(k * 16, 16)]
            c16 = cb[pl.ds(k * 16, 16)]
            w16 = wb[pl.ds(k * 16, 16)]
            dr = plsc.load_gather(disv, [r16])
            dc = plsc.load_gather(disv, [c16])
            nb[pl.ds(k * 16, 16)] = dr * w16 * dc
        pltpu.async_copy(nb, norm_ref.at[pl.ds(cid * 128, 128)], semN)
        nxt = jnp.minimum(cid + 2, last)
        pltpu.async_copy(chunk_of(rows_ref, nxt), rb, semE)
        pltpu.async_copy(chunk_of(cols_ref, nxt), cb, semE)
        pltpu.async_copy(chunk_of(ews_ref, nxt), wb, semE)

    def p3(j):
        a = base + 2 * j
        p3half(j, a, rbA, cbA, wbA, nbA, semEA)
        p3half(j, a + 1, rbB, cbB, wbB, nbB, semEB)
    _fori(CPT2 // 2, p3)
    pltpu.make_async_copy(chunk_of(rows_ref, 0), rbA, semEA).wait()
    pltpu.make_async_copy(chunk_of(cols_ref, 0), cbA, semEA).wait()
    pltpu.make_async_copy(chunk_of(ews_ref, 0), wbA, semEA).wait()
    pltpu.make_async_copy(chunk_of(rows_ref, 0), rbB, semEB).wait()
    pltpu.make_async_copy(chunk_of(cols_ref, 0), cbB, semEB).wait()
    pltpu.make_async_copy(chunk_of(ews_ref, 0), wbB, semEB).wait()
    pltpu.make_async_copy(nbA, norm_ref.at[pl.ds(0, 128)], semN).wait()
    pltpu.make_async_copy(nbB, norm_ref.at[pl.ds(0, 128)], semN).wait()


def _s1b(rows, cols, ews, pdeg):
    return pl.kernel(
        _s1b_body,
        out_type=(jax.ShapeDtypeStruct((EP,), jnp.float32),
                  jax.ShapeDtypeStruct((NP4, 128), jnp.float32)),
        mesh=plsc.VectorSubcoreMesh(core_axis_name="c", subcore_axis_name="s"),
        compiler_params=pltpu.CompilerParams(needs_layout_passes=False, use_tc_tiling_on_sc=False),
        scratch_types=[
            pltpu.VMEM((128,), jnp.int32),
            pltpu.VMEM((128,), jnp.int32),
            pltpu.VMEM((128,), jnp.int32),
            pltpu.VMEM((128,), jnp.int32),
            pltpu.VMEM((128,), jnp.float32),
            pltpu.VMEM((128,), jnp.float32),
            pltpu.VMEM((128,), jnp.float32),
            pltpu.VMEM((128,), jnp.float32),
            pltpu.VMEM((32, 128), jnp.float32),
            pltpu.VMEM((SLICE,), jnp.float32),
            pltpu.VMEM((SLICE,), jnp.float32),
            pltpu.VMEM((NP,), jnp.float32),
            pltpu.VMEM_SHARED((NP,), jnp.float32),
            pltpu.SemaphoreType.DMA,
            pltpu.SemaphoreType.DMA,
            pltpu.SemaphoreType.DMA,
        ],
    )(rows, cols, ews, pdeg)


_DEPTH = 5


def _mp_body(t_ref, rows_ref, cols_ref, norm_ref, out_ref, *args):
    bufs = args[:6 * _DEPTH]
    zbuf = args[6 * _DEPTH]
    acc_sh = args[6 * _DEPTH + 1]
    sems = args[6 * _DEPTH + 2:]
    semZ = sems[3 * _DEPTH]
    c = lax.axis_index("c")
    w = lax.axis_index("s")
    coff = c * NP
    zero16 = jnp.zeros((16,), jnp.float32)
    base = w * CPT
    last = base + CPT - 1

    def bufset(i):
        return bufs[6 * i:6 * i + 6]

    def semset(i):
        return sems[3 * i:3 * i + 3]

    def chunk_of(ref, cid):
        return ref.at[pl.ds(cid * 128, 128)]

    def n_slice(cid):
        return norm_ref.at[pl.ds(cid * 128, 128)]

    for e in range(128):
        zbuf[e, pl.ds(0, 16)] = zero16
        zbuf[e, pl.ds(16, 16)] = zero16
    zd = [pltpu.async_copy(zbuf, acc_sh.at[pl.ds(w * SLICE + j * 128, 128)], semZ)
          for j in range(SLICE // 128)]
    for d in zd:
        d.wait()
    plsc.subcore_barrier()

    for i in range(_DEPTH):
        rb, cb, nb, gx, cx, gb = bufset(i)
        semE, semG, semS = semset(i)
        pltpu.async_copy(chunk_of(rows_ref, base + i), rb, semE)
        pltpu.async_copy(chunk_of(cols_ref, base + i), cb, semE)
        pltpu.async_copy(n_slice(base + i), nb, semE)

    def arrive(j, cid, rb, cb, nb, gx, cx, gb, semE, semG, semS):
        pltpu.make_async_copy(chunk_of(rows_ref, cid), rb, semE).wait()
        pltpu.make_async_copy(chunk_of(cols_ref, cid), cb, semE).wait()
        pltpu.make_async_copy(n_slice(cid), nb, semE).wait()

        @pl.when(j > 0)
        def _():
            pltpu.make_async_copy(gb, acc_sh.at[cx], semS).wait()
        for k in range(8):
            gx[pl.ds(k * 16, 16)] = rb[pl.ds(k * 16, 16)] + coff
            cx[pl.ds(k * 16, 16)] = cb[pl.ds(k * 16, 16)]
        pltpu.async_copy(t_ref.at[gx], gb, semG)
        nxt = jnp.minimum(cid + _DEPTH, last)
        pltpu.async_copy(chunk_of(rows_ref, nxt), rb, semE)
        pltpu.async_copy(chunk_of(cols_ref, nxt), cb, semE)

    def scale_scatter(cid, rb, cb, nb, gx, cx, gb, semE, semG, semS):
        pltpu.make_async_copy(t_ref.at[gx], gb, semG).wait()
        for k in range(8):
            n16 = nb[pl.ds(k * 16, 16)]
            for m in range(16):
                e = k * 16 + m
                sv = n16[m]
                gb[e, pl.ds(0, 16)] = gb[e, pl.ds(0, 16)] * sv
                gb[e, pl.ds(16, 16)] = gb[e, pl.ds(16, 16)] * sv
        pltpu.async_copy(gb, acc_sh.at[cx], semS, add=True)
        pltpu.async_copy(n_slice(jnp.minimum(cid + _DEPTH, last)), nb, semE)

    def body(j):
        a = base + _DEPTH * j
        for i in range(_DEPTH):
            arrive(j, a + i, *bufset(i), *semset(i))
        for i in range(_DEPTH):
            scale_scatter(a + i, *bufset(i), *semset(i))
    _fori(CPT // _DEPTH, body)

    for i in range(_DEPTH):
        rb, cb, nb, gx, cx, gb = bufset(i)
        semE, semG, semS = semset(i)
        pltpu.make_async_copy(chunk_of(rows_ref, 0), rb, semE).wait()
        pltpu.make_async_copy(chunk_of(cols_ref, 0), cb, semE).wait()
        pltpu.make_async_copy(n_slice(0), nb, semE).wait()
        pltpu.make_async_copy(gb, acc_sh.at[cx], semS).wait()
    plsc.subcore_barrier()

    pltpu.sync_copy(acc_sh.at[pl.ds(w * SLICE, SLICE)],
                    out_ref.at[c, pl.ds(w * SLICE, SLICE)])


def _mp(t_flat, rows, cols, norm):
    bufsets = []
    for _ in range(_DEPTH):
        bufsets += [
            pltpu.VMEM((128,), jnp.int32),
            pltpu.VMEM((128,), jnp.int32),
            pltpu.VMEM((128,), jnp.float32),
            pltpu.VMEM((128,), jnp.int32),
            pltpu.VMEM((128,), jnp.int32),
            pltpu.VMEM((128, F), jnp.float32),
        ]
    sems = [pltpu.SemaphoreType.DMA] * (3 * _DEPTH + 1)
    return pl.kernel(
        _mp_body,
        out_type=jax.ShapeDtypeStruct((2, NP, F), jnp.float32),
        mesh=plsc.VectorSubcoreMesh(core_axis_name="c", subcore_axis_name="s"),
        compiler_params=pltpu.CompilerParams(needs_layout_passes=False, use_tc_tiling_on_sc=False),
        scratch_types=(bufsets
                       + [pltpu.VMEM((128, F), jnp.float32),
                          pltpu.VMEM_SHARED((NP, F), jnp.float32)]
                       + sems),
    )(t_flat, rows, cols, norm)


def kernel(x, edge_index, batch, global_features, edge_weight,
           W_emb, b_emb, W1, b1, W2, b2, Wg, bg, Wh1, bh1, Wh2, bh2):
    x_pad = jnp.pad(x, ((0, NP - N), (0, 0)))
    x4 = x_pad.reshape(NP4, 4 * VIN)
    batch_pad = jnp.pad(batch, (0, NP - N), constant_values=B)
    batch3 = batch_pad.reshape(NBLK, R4, 4).transpose(0, 2, 1)

    def bd4(w):
        return jax.scipy.linalg.block_diag(w, w, w, w)

    BDWe = bd4(W_emb.T)
    BDW1a = bd4(W1[:F, :].T)
    BDW1b = bd4(W1[F:, :].T)
    BDW2a = bd4(W2[:F, :].T)
    BDW2b = bd4(W2[F:, :].T)
    be4 = jnp.tile(b_emb, 4).reshape(1, 4 * H)
    b1_4 = jnp.tile(b1, 4).reshape(1, 4 * H)
    b2_4 = jnp.tile(b2, 4).reshape(1, 4 * H)
    pad_e = EP - E
    pad_idx = (jnp.arange(pad_e, dtype=jnp.int32) * 997) % N
    rows = jnp.concatenate([edge_index[0], pad_idx])
    cols = jnp.concatenate([edge_index[1], pad_idx])
    ews = jnp.concatenate([edge_weight, jnp.zeros((pad_e,), jnp.float32)])

    bg_r = bg.reshape(1, H)
    bh1_r = bh1.reshape(1, H)
    bh2_r = bh2.reshape(1, A)

    t1 = _d1(x4, BDWe, be4, BDW1a, BDW1b)
    pdeg = _s1a(cols, ews)
    norm, dissq = _s1b(rows, cols, ews, pdeg)
    acc1 = _mp(t1.reshape(2 * NP, F), rows, cols, norm)
    t2 = _d2(acc1.reshape(2, NP4, 128), t1, dissq, b1_4, BDW2a, BDW2b)
    acc2 = _mp(t2.reshape(2 * NP, F), rows, cols, norm)
    q = _d3(acc2.reshape(2, NP4, 128), t2, dissq, b2_4, batch3,
            global_features, Wg, bg_r, Wh1, bh1_r, Wh2, bh2_r)
    return q


# speedup vs baseline: 1.3303x; 1.0326x over previous
import jax
import jax.numpy as jnp
from jax import lax
from jax.experimental import pallas as pl
from jax.experimental.pallas import tpu as pltpu
from jax.experimental.pallas import tpu_sc as plsc

N = 50000
E = 800000
H = 64
VIN = 4
GFS = 16
A = 8
B = 64

NP = 51200
EP = 819200
EC = EP // 128
R = 6400
NBLK = NP // R
CPT = EC // 16
SLICE = NP // 16
F = 32


R4 = R // 4
NP4 = NP // 4


def _interleave(a4):
    parts = []
    for j in range(4):
        parts.append(a4[0][:, 32 * j:32 * j + 32])
        parts.append(a4[1][:, 32 * j:32 * j + 32])
    return jnp.concatenate(parts, axis=1)


def _d1_body(x_ref, we_ref, be_ref, w1a_ref, w1b_ref, out_ref):
    xb = x_ref[...]
    h = jnp.dot(xb, we_ref[...], preferred_element_type=jnp.float32)
    h = jnp.maximum(h + be_ref[...], 0.0)
    t0 = jnp.dot(h, w1a_ref[...], preferred_element_type=jnp.float32)
    t1 = jnp.dot(h, w1b_ref[...], preferred_element_type=jnp.float32)
    out_ref[...] = jnp.stack([t0, t1], axis=0)


def _d1(x4, BDWe, be4, BDW1a, BDW1b):
    return pl.pallas_call(
        _d1_body,
        grid=(NBLK,),
        in_specs=[
            pl.BlockSpec((R4, 4 * VIN), lambda i: (i, 0)),
            pl.BlockSpec((4 * VIN, 4 * H), lambda i: (0, 0)),
            pl.BlockSpec((1, 4 * H), lambda i: (0, 0)),
            pl.BlockSpec((4 * H, 128), lambda i: (0, 0)),
            pl.BlockSpec((4 * H, 128), lambda i: (0, 0)),
        ],
        out_specs=pl.BlockSpec((2, R4, 128), lambda i: (0, i, 0)),
        out_shape=jax.ShapeDtypeStruct((2, NP4, 128), jnp.float32),
    )(x4, BDWe, be4, BDW1a, BDW1b)


def _d2_body(acc_ref, t_ref, dsq_ref, b_ref, w2a_ref, w2b_ref, out_ref):
    a4 = acc_ref[...] + t_ref[...] * dsq_ref[...][None]
    h = _interleave(a4) + b_ref[...]
    h = jnp.maximum(h, 0.0)
    t0 = jnp.dot(h, w2a_ref[...], preferred_element_type=jnp.float32)
    t1 = jnp.dot(h, w2b_ref[...], preferred_element_type=jnp.float32)
    out_ref[...] = jnp.stack([t0, t1], axis=0)


def _d2(acc1, t1, dissq, b4, BDW2a, BDW2b):
    return pl.pallas_call(
        _d2_body,
        grid=(NBLK,),
        in_specs=[
            pl.BlockSpec((2, R4, 128), lambda i: (0, i, 0)),
            pl.BlockSpec((2, R4, 128), lambda i: (0, i, 0)),
            pl.BlockSpec((R4, 128), lambda i: (i, 0)),
            pl.BlockSpec((1, 4 * H), lambda i: (0, 0)),
            pl.BlockSpec((4 * H, 128), lambda i: (0, 0)),
            pl.BlockSpec((4 * H, 128), lambda i: (0, 0)),
        ],
        out_specs=pl.BlockSpec((2, R4, 128), lambda i: (0, i, 0)),
        out_shape=jax.ShapeDtypeStruct((2, NP4, 128), jnp.float32),
    )(acc1, t1, dissq, b4, BDW2a, BDW2b)


def _d3_body(acc_ref, t_ref, dsq_ref, b2_ref, batch_ref, gf_ref, wg_ref,
             bg_ref, wh1_ref, bh1_ref, wh2_ref, bh2_ref, q_ref, sums_ref,
             cnt_ref):
    i = pl.program_id(0)

    @pl.when(i == 0)
    def _init():
        sums_ref[...] = jnp.zeros_like(sums_ref)
        cnt_ref[...] = jnp.zeros_like(cnt_ref)

    a4 = acc_ref[...] + t_ref[...] * dsq_ref[...][None]
    h = _interleave(a4) + b2_ref[...]
    h = jnp.maximum(h, 0.0)
    bt4 = batch_ref[0]
    ids = lax.broadcasted_iota(jnp.int32, (B, R4), 0)
    for j in range(4):
        onehot = (bt4[j][None, :] == ids).astype(jnp.float32)
        sums_ref[...] += jnp.dot(onehot, h[:, H * j:H * j + H],
                                 preferred_element_type=jnp.float32)
        cnt_ref[...] += jnp.sum(onehot, axis=1, keepdims=True)

    @pl.when(i == NBLK - 1)
    def _head():
        ge = sums_ref[...] / jnp.maximum(cnt_ref[...], 1.0)
        glob = jnp.dot(gf_ref[...], wg_ref[...].T, preferred_element_type=jnp.float32)
        glob = jnp.maximum(glob + bg_ref[...], 0.0)
        wh1 = wh1_ref[...]
        hid = (jnp.dot(ge, wh1[:, :H].T, preferred_element_type=jnp.float32)
               + jnp.dot(glob, wh1[:, H:].T, preferred_element_type=jnp.float32)
               + bh1_ref[...])
        hid = jnp.maximum(hid, 0.0)
        q_ref[...] = (jnp.dot(hid, wh2_ref[...].T, preferred_element_type=jnp.float32)
                      + bh2_ref[...])


def _d3(acc2, t2, dissq, b2, batch3, gf, Wg, bg, Wh1, bh1, Wh2, bh2):
    return pl.pallas_call(
        _d3_body,
        grid=(NBLK,),
        in_specs=[
            pl.BlockSpec((2, R4, 128), lambda i: (0, i, 0)),
            pl.BlockSpec((2, R4, 128), lambda i: (0, i, 0)),
            pl.BlockSpec((R4, 128), lambda i: (i, 0)),
            pl.BlockSpec((1, 4 * H), lambda i: (0, 0)),
            pl.BlockSpec((1, 4, R4), lambda i: (i, 0, 0)),
            pl.BlockSpec((B, GFS), lambda i: (0, 0)),
            pl.BlockSpec((H, GFS), lambda i: (0, 0)),
            pl.BlockSpec((1, H), lambda i: (0, 0)),
            pl.BlockSpec((H, 2 * H), lambda i: (0, 0)),
            pl.BlockSpec((1, H), lambda i: (0, 0)),
            pl.BlockSpec((A, H), lambda i: (0, 0)),
            pl.BlockSpec((1, A), lambda i: (0, 0)),
        ],
        out_specs=pl.BlockSpec((B, A), lambda i: (0, 0)),
        out_shape=jax.ShapeDtypeStruct((B, A), jnp.float32),
        scratch_shapes=[
            pltpu.VMEM((B, H), jnp.float32),
            pltpu.VMEM((B, 1), jnp.float32),
        ],
    )(acc2, t2, dissq, b2, batch3, gf, Wg, bg, Wh1, bh1, Wh2, bh2)


def _fori(n, body):
    lax.fori_loop(0, n, lambda i, c: (body(i), 0)[1], 0)


CPT2 = EC // 32


def _s1a_body(cols_ref, ews_ref, pdeg_ref,
              cbA, cbB, wbA, wbB, cxA, cxB, sxA, sxB, dslice, deg_sh,
              semEA, semEB, semSA, semSB):
    c = lax.axis_index("c")
    w = lax.axis_index("s")
    zero16 = jnp.zeros((16,), jnp.float32)
    base = c * (EC // 2) + w * CPT2
    last = base + CPT2 - 1

    def chunk_of(ref, cid):
        return ref.at[pl.ds(cid * 128, 128)]

    def zrow(k):
        dslice[pl.ds(k * 16, 16)] = zero16
    _fori(SLICE // 16, zrow)
    pltpu.sync_copy(dslice, deg_sh.at[pl.ds(w * SLICE, SLICE)])
    plsc.subcore_barrier()

    pltpu.async_copy(chunk_of(cols_ref, base), cbA, semEA)
    pltpu.async_copy(chunk_of(ews_ref, base), wbA, semEA)
    pltpu.async_copy(chunk_of(cols_ref, base + 1), cbB, semEB)
    pltpu.async_copy(chunk_of(ews_ref, base + 1), wbB, semEB)

    def p1half(j, cid, cb, wb, cx, sx, semE, semS):
        pltpu.make_async_copy(chunk_of(cols_ref, cid), cb, semE).wait()
        pltpu.make_async_copy(chunk_of(ews_ref, cid), wb, semE).wait()

        @pl.when(j > 0)
        def _():
            pltpu.make_async_copy(sx, deg_sh.at[cx], semS).wait()
        for k in range(8):
            cx[pl.ds(k * 16, 16)] = cb[pl.ds(k * 16, 16)]
            sx[pl.ds(k * 16, 16)] = wb[pl.ds(k * 16, 16)]
        pltpu.async_copy(sx, deg_sh.at[cx], semS, add=True)
        nxt = jnp.minimum(cid + 2, last)
        pltpu.async_copy(chunk_of(cols_ref, nxt), cb, semE)
        pltpu.async_copy(chunk_of(ews_ref, nxt), wb, semE)

    def p1(j):
        a = base + 2 * j
        p1half(j, a, cbA, wbA, cxA, sxA, semEA, semSA)
        p1half(j, a + 1, cbB, wbB, cxB, sxB, semEB, semSB)
    _fori(CPT2 // 2, p1)
    pltpu.make_async_copy(chunk_of(cols_ref, 0), cbA, semEA).wait()
    pltpu.make_async_copy(chunk_of(ews_ref, 0), wbA, semEA).wait()
    pltpu.make_async_copy(chunk_of(cols_ref, 0), cbB, semEB).wait()
    pltpu.make_async_copy(chunk_of(ews_ref, 0), wbB, semEB).wait()
    pltpu.make_async_copy(sxA, deg_sh.at[cxA], semSA).wait()
    pltpu.make_async_copy(sxB, deg_sh.at[cxB], semSB).wait()
    plsc.subcore_barrier()
    pltpu.sync_copy(deg_sh.at[pl.ds(w * SLICE, SLICE)],
                    pdeg_ref.at[c, pl.ds(w * SLICE, SLICE)])


def _s1a(cols, ews):
    return pl.kernel(
        _s1a_body,
        out_type=jax.ShapeDtypeStruct((2, NP), jnp.float32),
        mesh=plsc.VectorSubcoreMesh(core_axis_name="c", subcore_axis_name="s"),
        compiler_params=pltpu.CompilerParams(needs_layout_passes=False, use_tc_tiling_on_sc=False),
        scratch_types=[
            pltpu.VMEM((128,), jnp.int32),
            pltpu.VMEM((128,), jnp.int32),
            pltpu.VMEM((128,), jnp.float32),
            pltpu.VMEM((128,), jnp.float32),
            pltpu.VMEM((128,), jnp.int32),
            pltpu.VMEM((128,), jnp.int32),
            pltpu.VMEM((128,), jnp.float32),
            pltpu.VMEM((128,), jnp.float32),
            pltpu.VMEM((SLICE,), jnp.float32),
            pltpu.VMEM_SHARED((NP,), jnp.float32),
            pltpu.SemaphoreType.DMA,
            pltpu.SemaphoreType.DMA,
            pltpu.SemaphoreType.DMA,
            pltpu.SemaphoreType.DMA,
        ],
    )(cols, ews)


_S1D = 4


def _s1b_body(rows_ref, cols_ref, ews_ref, pdeg_ref, norm_ref, dsq_ref, *args):
    bufs = args[:4 * _S1D]
    d2b, dslice, pdg2, disv, deg_sh = args[4 * _S1D:4 * _S1D + 5]
    semEs = args[4 * _S1D + 5:4 * _S1D + 5 + _S1D]
    semN = args[4 * _S1D + 5 + _S1D]
    c = lax.axis_index("c")
    w = lax.axis_index("s")
    zero16 = jnp.zeros((16,), jnp.float32)
    base = c * (EC // 2) + w * CPT2
    last = base + CPT2 - 1

    def chunk_of(ref, cid):
        return ref.at[pl.ds(cid * 128, 128)]

    pltpu.sync_copy(pdeg_ref.at[0, pl.ds(w * SLICE, SLICE)], dslice)
    pltpu.sync_copy(pdeg_ref.at[1, pl.ds(w * SLICE, SLICE)], pdg2)
    magic = jnp.int32(0x5F3759DF)

    def rsq(k):
        d = dslice[pl.ds(k * 16, 16)] + pdg2[pl.ds(k * 16, 16)] + 1.0
        bits = lax.bitcast_convert_type(d, jnp.int32)
        y = lax.bitcast_convert_type(magic - lax.shift_right_logical(bits, 1), jnp.float32)
        hd = 0.5 * d
        for _ in range(3):
            y = y * (1.5 - hd * y * y)
        dslice[pl.ds(k * 16, 16)] = y
    _fori(SLICE // 16, rsq)
    pltpu.sync_copy(dslice, deg_sh.at[pl.ds(w * SLICE, SLICE)])

    @pl.when(c == 0)
    def _d2emit():
        def d2chunk(j):
            for k in range(8):
                d16 = dslice[pl.ds(j * 128 + k * 16, 16)]
                s16 = d16 * d16
                for m in range(16):
                    e = k * 16 + m
                    sv = s16[m]
                    d2b[e // 4, pl.ds((e % 4) * 32, 16)] = zero16 + sv
                    d2b[e // 4, pl.ds((e % 4) * 32 + 16, 16)] = zero16 + sv
            pltpu.sync_copy(d2b, dsq_ref.at[pl.ds(w * (SLICE // 4) + j * 32, 32)])
        _fori(SLICE // 128, d2chunk)
    plsc.subcore_barrier()

    for i in range(_S1D):
        rb, cb, wb, nb = bufs[4 * i:4 * i + 4]
        pltpu.async_copy(chunk_of(rows_ref, base + i), rb, semEs[i])
        pltpu.async_copy(chunk_of(cols_ref, base + i), cb, semEs[i])
        pltpu.async_copy(chunk_of(ews_ref, base + i), wb, semEs[i])
    pltpu.sync_copy(deg_sh, disv)

    def p3set(j, cid, rb, cb, wb, nb, semE):
        pltpu.make_async_copy(chunk_of(rows_ref, cid), rb, semE).wait()
        pltpu.make_async_copy(chunk_of(cols_ref, cid), cb, semE).wait()
        pltpu.make_async_copy(chunk_of(ews_ref, cid), wb, semE).wait()

        @pl.when(j > 0)
        def _():
            pltpu.make_async_copy(nb, norm_ref.at[pl.ds(0, 128)], semN).wait()
        for k in range(8):
            r16 = rb[pl.ds(k * 16, 16)]
            c16 = cb[pl.ds(k * 16, 16)]
            w16 = wb[pl.ds(k * 16, 16)]
            dr = plsc.load_gather(disv, [r16])
            dc = plsc.load_gather(disv, [c16])
            nb[pl.ds(k * 16, 16)] = dr * w16 * dc
        pltpu.async_copy(nb, norm_ref.at[pl.ds(cid * 128, 128)], semN)
        nxt = jnp.minimum(cid + _S1D, last)
        pltpu.async_copy(chunk_of(rows_ref, nxt), rb, semE)
        pltpu.async_copy(chunk_of(cols_ref, nxt), cb, semE)
        pltpu.async_copy(chunk_of(ews_ref, nxt), wb, semE)

    def p3(j):
        a = base + _S1D * j
        for i in range(_S1D):
            p3set(j, a + i, *bufs[4 * i:4 * i + 4], semEs[i])
    _fori(CPT2 // _S1D, p3)
    for i in range(_S1D):
        rb, cb, wb, nb = bufs[4 * i:4 * i + 4]
        pltpu.make_async_copy(chunk_of(rows_ref, 0), rb, semEs[i]).wait()
        pltpu.make_async_copy(chunk_of(cols_ref, 0), cb, semEs[i]).wait()
        pltpu.make_async_copy(chunk_of(ews_ref, 0), wb, semEs[i]).wait()
        pltpu.make_async_copy(nb, norm_ref.at[pl.ds(0, 128)], semN).wait()


def _s1b(rows, cols, ews, pdeg):
    bufsets = []
    for _ in range(_S1D):
        bufsets += [
            pltpu.VMEM((128,), jnp.int32),
            pltpu.VMEM((128,), jnp.int32),
            pltpu.VMEM((128,), jnp.float32),
            pltpu.VMEM((128,), jnp.float32),
        ]
    return pl.kernel(
        _s1b_body,
        out_type=(jax.ShapeDtypeStruct((EP,), jnp.float32),
                  jax.ShapeDtypeStruct((NP4, 128), jnp.float32)),
        mesh=plsc.VectorSubcoreMesh(core_axis_name="c", subcore_axis_name="s"),
        compiler_params=pltpu.CompilerParams(needs_layout_passes=False, use_tc_tiling_on_sc=False),
        scratch_types=(bufsets
                       + [pltpu.VMEM((32, 128), jnp.float32),
                          pltpu.VMEM((SLICE,), jnp.float32),
                          pltpu.VMEM((SLICE,), jnp.float32),
                          pltpu.VMEM((NP,), jnp.float32),
                          pltpu.VMEM_SHARED((NP,), jnp.float32)]
                       + [pltpu.SemaphoreType.DMA] * _S1D
                       + [pltpu.SemaphoreType.DMA]),
    )(rows, cols, ews, pdeg)


_DEPTH = 5


def _mp_body(t_ref, rows_ref, cols_ref, norm_ref, out_ref, *args):
    bufs = args[:6 * _DEPTH]
    zbuf = args[6 * _DEPTH]
    acc_sh = args[6 * _DEPTH + 1]
    sems = args[6 * _DEPTH + 2:]
    semZ = sems[3 * _DEPTH]
    c = lax.axis_index("c")
    w = lax.axis_index("s")
    coff = c * NP
    zero16 = jnp.zeros((16,), jnp.float32)
    base = w * CPT
    last = base + CPT - 1

    def bufset(i):
        return bufs[6 * i:6 * i + 6]

    def semset(i):
        return sems[3 * i:3 * i + 3]

    def chunk_of(ref, cid):
        return ref.at[pl.ds(cid * 128, 128)]

    def n_slice(cid):
        return norm_ref.at[pl.ds(cid * 128, 128)]

    for e in range(128):
        zbuf[e, pl.ds(0, 16)] = zero16
        zbuf[e, pl.ds(16, 16)] = zero16
    zd = [pltpu.async_copy(zbuf, acc_sh.at[pl.ds(w * SLICE + j * 128, 128)], semZ)
          for j in range(SLICE // 128)]
    for d in zd:
        d.wait()
    plsc.subcore_barrier()

    for i in range(_DEPTH):
        rb, cb, nb, gx, cx, gb = bufset(i)
        semE, semG, semS = semset(i)
        pltpu.async_copy(chunk_of(rows_ref, base + i), rb, semE)
        pltpu.async_copy(chunk_of(cols_ref, base + i), cb, semE)
        pltpu.async_copy(n_slice(base + i), nb, semE)

    def arrive(j, cid, rb, cb, nb, gx, cx, gb, semE, semG, semS):
        pltpu.make_async_copy(chunk_of(rows_ref, cid), rb, semE).wait()
        pltpu.make_async_copy(chunk_of(cols_ref, cid), cb, semE).wait()
        pltpu.make_async_copy(n_slice(cid), nb, semE).wait()

        @pl.when(j > 0)
        def _():
            pltpu.make_async_copy(gb, acc_sh.at[cx], semS).wait()
        for k in range(8):
            gx[pl.ds(k * 16, 16)] = rb[pl.ds(k * 16, 16)] + coff
            cx[pl.ds(k * 16, 16)] = cb[pl.ds(k * 16, 16)]
        pltpu.async_copy(t_ref.at[gx], gb, semG)
        nxt = jnp.minimum(cid + _DEPTH, last)
        pltpu.async_copy(chunk_of(rows_ref, nxt), rb, semE)
        pltpu.async_copy(chunk_of(cols_ref, nxt), cb, semE)

    def scale_scatter(cid, rb, cb, nb, gx, cx, gb, semE, semG, semS):
        pltpu.make_async_copy(t_ref.at[gx], gb, semG).wait()
        for k in range(8):
            n16 = nb[pl.ds(k * 16, 16)]
            for m in range(16):
                e = k * 16 + m
                sv = n16[m]
                gb[e, pl.ds(0, 16)] = gb[e, pl.ds(0, 16)] * sv
                gb[e, pl.ds(16, 16)] = gb[e, pl.ds(16, 16)] * sv
        pltpu.async_copy(gb, acc_sh.at[cx], semS, add=True)
        pltpu.async_copy(n_slice(jnp.minimum(cid + _DEPTH, last)), nb, semE)

    def body(j):
        a = base + _DEPTH * j
        for i in range(_DEPTH):
            arrive(j, a + i, *bufset(i), *semset(i))
        for i in range(_DEPTH):
            scale_scatter(a + i, *bufset(i), *semset(i))
    _fori(CPT // _DEPTH, body)

    for i in range(_DEPTH):
        rb, cb, nb, gx, cx, gb = bufset(i)
        semE, semG, semS = semset(i)
        pltpu.make_async_copy(chunk_of(rows_ref, 0), rb, semE).wait()
        pltpu.make_async_copy(chunk_of(cols_ref, 0), cb, semE).wait()
        pltpu.make_async_copy(n_slice(0), nb, semE).wait()
        pltpu.make_async_copy(gb, acc_sh.at[cx], semS).wait()
    plsc.subcore_barrier()

    pltpu.sync_copy(acc_sh.at[pl.ds(w * SLICE, SLICE)],
                    out_ref.at[c, pl.ds(w * SLICE, SLICE)])


def _mp(t_flat, rows, cols, norm):
    bufsets = []
    for _ in range(_DEPTH):
        bufsets += [
            pltpu.VMEM((128,), jnp.int32),
            pltpu.VMEM((128,), jnp.int32),
            pltpu.VMEM((128,), jnp.float32),
            pltpu.VMEM((128,), jnp.int32),
            pltpu.VMEM((128,), jnp.int32),
            pltpu.VMEM((128, F), jnp.float32),
        ]
    sems = [pltpu.SemaphoreType.DMA] * (3 * _DEPTH + 1)
    return pl.kernel(
        _mp_body,
        out_type=jax.ShapeDtypeStruct((2, NP, F), jnp.float32),
        mesh=plsc.VectorSubcoreMesh(core_axis_name="c", subcore_axis_name="s"),
        compiler_params=pltpu.CompilerParams(needs_layout_passes=False, use_tc_tiling_on_sc=False),
        scratch_types=(bufsets
                       + [pltpu.VMEM((128, F), jnp.float32),
                          pltpu.VMEM_SHARED((NP, F), jnp.float32)]
                       + sems),
    )(t_flat, rows, cols, norm)


def kernel(x, edge_index, batch, global_features, edge_weight,
           W_emb, b_emb, W1, b1, W2, b2, Wg, bg, Wh1, bh1, Wh2, bh2):
    x_pad = jnp.pad(x, ((0, NP - N), (0, 0)))
    x4 = x_pad.reshape(NP4, 4 * VIN)
    batch_pad = jnp.pad(batch, (0, NP - N), constant_values=B)
    batch3 = batch_pad.reshape(NBLK, R4, 4).transpose(0, 2, 1)

    def bd4(w):
        return jax.scipy.linalg.block_diag(w, w, w, w)

    BDWe = bd4(W_emb.T)
    BDW1a = bd4(W1[:F, :].T)
    BDW1b = bd4(W1[F:, :].T)
    BDW2a = bd4(W2[:F, :].T)
    BDW2b = bd4(W2[F:, :].T)
    be4 = jnp.tile(b_emb, 4).reshape(1, 4 * H)
    b1_4 = jnp.tile(b1, 4).reshape(1, 4 * H)
    b2_4 = jnp.tile(b2, 4).reshape(1, 4 * H)
    pad_e = EP - E
    pad_idx = (jnp.arange(pad_e, dtype=jnp.int32) * 997) % N
    rows = jnp.concatenate([edge_index[0], pad_idx])
    cols = jnp.concatenate([edge_index[1], pad_idx])
    ews = jnp.concatenate([edge_weight, jnp.zeros((pad_e,), jnp.float32)])

    bg_r = bg.reshape(1, H)
    bh1_r = bh1.reshape(1, H)
    bh2_r = bh2.reshape(1, A)

    t1 = _d1(x4, BDWe, be4, BDW1a, BDW1b)
    pdeg = _s1a(cols, ews)
    norm, dissq = _s1b(rows, cols, ews, pdeg)
    acc1 = _mp(t1.reshape(2 * NP, F), rows, cols, norm)
    t2 = _d2(acc1.reshape(2, NP4, 128), t1, dissq, b1_4, BDW2a, BDW2b)
    acc2 = _mp(t2.reshape(2 * NP, F), rows, cols, norm)
    q = _d3(acc2.reshape(2, NP4, 128), t2, dissq, b2_4, batch3,
            global_features, Wg, bg_r, Wh1, bh1_r, Wh2, bh2_r)
    return q
